# baseline (speedup 1.0000x reference)
import functools

import jax
import jax.numpy as jnp
from jax import lax
from jax.experimental import pallas as pl
from jax.experimental.pallas import tpu as pltpu
from jax.experimental.pallas import tpu_sc as plsc

_NW = 32
_K = 128


def _embed_lookup(idx, weight, steps):
    B = _NW * steps * _K
    D = weight.shape[1]
    mesh = plsc.VectorSubcoreMesh(core_axis_name="c", subcore_axis_name="s")

    @functools.partial(
        pl.kernel,
        mesh=mesh,
        compiler_params=pltpu.CompilerParams(use_tc_tiling_on_sc=False),
        out_type=jax.ShapeDtypeStruct((B, D), jnp.float32),
        scratch_types=[
            pltpu.VMEM((steps, _K), jnp.int32),
            pltpu.VMEM((_K, D), jnp.float32),
            pltpu.SemaphoreType.DMA,
        ],
    )
    def k(idx_hbm, w_hbm, out_hbm, idx_v, rows_v, sem):
        wid = lax.axis_index("s") * 2 + lax.axis_index("c")
        pltpu.sync_copy(idx_hbm.at[wid], idx_v)
        base = wid * (steps * _K)

        def body(j, carry):
            pltpu.async_copy(w_hbm.at[idx_v.at[j]], rows_v, sem).wait()
            pltpu.sync_copy(rows_v, out_hbm.at[pl.ds(base + j * _K, _K)])
            return carry

        lax.fori_loop(0, steps, body, 0)

    return k(idx, weight)


def kernel(tokens, weight):
    S, T = tokens.shape
    D = weight.shape[1]
    B = S * T
    steps = B // (_NW * _K)
    idx = tokens.reshape(_NW, steps, _K).astype(jnp.int32)
    out = _embed_lookup(idx, weight, steps)
    return out.reshape(S, T, D)


# speedup vs baseline: 1.1195x; 1.1195x over previous
import functools

import jax
import jax.numpy as jnp
from jax import lax
from jax.experimental import pallas as pl
from jax.experimental.pallas import tpu as pltpu
from jax.experimental.pallas import tpu_sc as plsc

_NW = 32
_K = 128
_SLOTS = 8
_DEPTH = 4


def _embed_lookup(idx, weight, steps):
    B = _NW * steps * _K
    D = weight.shape[1]
    mesh = plsc.VectorSubcoreMesh(core_axis_name="c", subcore_axis_name="s")

    @functools.partial(
        pl.kernel,
        mesh=mesh,
        compiler_params=pltpu.CompilerParams(use_tc_tiling_on_sc=False),
        out_type=jax.ShapeDtypeStruct((B, D), jnp.float32),
        scratch_types=[
            pltpu.VMEM((steps, _K), jnp.int32),
            pltpu.VMEM((_SLOTS, _K, D), jnp.float32),
        ]
        + [pltpu.SemaphoreType.DMA] * (2 * _SLOTS),
    )
    def k(idx_hbm, w_hbm, out_hbm, idx_v, rows_v, *sems):
        gsem, wsem = sems[:_SLOTS], sems[_SLOTS:]
        wid = lax.axis_index("s") * 2 + lax.axis_index("c")
        pltpu.sync_copy(idx_hbm.at[wid], idx_v)
        base = wid * (steps * _K)

        def fire_gather(j, b):
            pltpu.async_copy(w_hbm.at[idx_v.at[j]], rows_v.at[b], gsem[b])

        for b in range(_DEPTH):
            fire_gather(b, b)

        def outer(j0, carry):
            for b in range(_SLOTS):
                j = j0 * _SLOTS + b
                pltpu.make_async_copy(
                    w_hbm.at[idx_v.at[j]], rows_v.at[b], gsem[b]
                ).wait()
                pltpu.async_copy(
                    rows_v.at[b], out_hbm.at[pl.ds(base + j * _K, _K)], wsem[b]
                )
                jn = j + _DEPTH
                bn = (b + _DEPTH) % _SLOTS

                @pl.when(jn < steps)
                def _():
                    @pl.when(jn >= _SLOTS)
                    def _():
                        pltpu.make_async_copy(
                            rows_v.at[bn],
                            out_hbm.at[pl.ds(base + (jn - _SLOTS) * _K, _K)],
                            wsem[bn],
                        ).wait()

                    fire_gather(jn, bn)

            return carry

        lax.fori_loop(0, steps // _SLOTS, outer, 0)
        for b in range(_SLOTS):
            pltpu.make_async_copy(
                rows_v.at[b], out_hbm.at[pl.ds(base, _K)], wsem[b]
            ).wait()

    return k(idx, weight)


def kernel(tokens, weight):
    S, T = tokens.shape
    D = weight.shape[1]
    B = S * T
    steps = B // (_NW * _K)
    idx = tokens.reshape(_NW, steps, _K).astype(jnp.int32)
    out = _embed_lookup(idx, weight, steps)
    return out.reshape(S, T, D)
